# baseline (speedup 1.0000x reference)
import functools

import jax
import jax.numpy as jnp
from jax import lax
from jax.experimental import pallas as pl
from jax.experimental.pallas import tpu as pltpu
from jax.experimental.pallas import tpu_sc as plsc

B = 16384
D = 64
NC = 2
NS = 16
NW = NC * NS
CHUNK = 128
BPW = B // NW
CPW = BPW // CHUNK
GPC = CHUNK // 16

_mesh = plsc.VectorSubcoreMesh(core_axis_name="c", subcore_axis_name="s")


@functools.partial(
    pl.kernel,
    out_type=jax.ShapeDtypeStruct((B,), jnp.float32),
    mesh=_mesh,
    compiler_params=pltpu.CompilerParams(needs_layout_passes=False,
                                         use_tc_tiling_on_sc=True),
    scratch_types=[
        pltpu.VMEM((CPW, CHUNK), jnp.int32),
        pltpu.VMEM((CPW, CHUNK), jnp.int32),
        pltpu.VMEM((CPW, CHUNK), jnp.int32),
        pltpu.VMEM((CPW, CHUNK), jnp.int32),
        pltpu.VMEM((CHUNK, 2 * D), jnp.float32),
        pltpu.VMEM((CHUNK, 2 * D), jnp.float32),
        pltpu.VMEM((BPW,), jnp.float32),
        pltpu.VMEM((BPW,), jnp.float32),
        pltpu.VMEM((BPW,), jnp.float32),
        pltpu.VMEM((16,), jnp.float32),
        pltpu.SemaphoreType.DMA,
        pltpu.SemaphoreType.DMA,
    ],
)
def _mf_kernel(uid_hbm, iid_hbm, uemb, iemb, ubias, ibias, gbias, out_hbm,
               uidx_v, iidx_v, uhalf_v, ihalf_v, urows, irows,
               ub_v, ib_v, out_v, gb_v, semg, semb):
    wid = lax.axis_index("s") * NC + lax.axis_index("c")
    crow0 = wid * CPW

    pltpu.sync_copy(uid_hbm.at[pl.ds(crow0, CPW)], uidx_v)
    pltpu.sync_copy(iid_hbm.at[pl.ds(crow0, CPW)], iidx_v)
    pltpu.sync_copy(gbias, gb_v)

    for c in range(CPW):
        for t in range(CHUNK // 16):
            s = pl.ds(t * 16, 16)
            uhalf_v[c, s] = lax.shift_right_logical(uidx_v[c, s], 1)
            ihalf_v[c, s] = lax.shift_right_logical(iidx_v[c, s], 1)

    bias_copies = []
    for c in range(CPW):
        dst = pl.ds(c * CHUNK, CHUNK)
        bias_copies.append(pltpu.make_async_copy(ubias.at[uidx_v.at[c]],
                                                 ub_v.at[dst], semb))
        bias_copies.append(pltpu.make_async_copy(ibias.at[iidx_v.at[c]],
                                                 ib_v.at[dst], semb))
    for cp in bias_copies:
        cp.start()

    lane = lax.iota(jnp.int32, 16)
    gval = gb_v[...]

    for c in range(CPW):
        cu = pltpu.make_async_copy(uemb.at[uhalf_v.at[c]], urows, semg)
        ci = pltpu.make_async_copy(iemb.at[ihalf_v.at[c]], irows, semg)
        cu.start()
        ci.start()
        cu.wait()
        ci.wait()
        if c == 0:
            for cp in bias_copies:
                cp.wait()

        def group_body(g, _, c=c):
            rows = g * 16 + lane
            ids_u = uidx_v[c, pl.ds(g * 16, 16)]
            ids_i = iidx_v[c, pl.ds(g * 16, 16)]
            par_u = (ids_u & 1) * D
            par_i = (ids_i & 1) * D
            b0 = c * CHUNK + g * 16
            acc = ub_v[pl.ds(b0, 16)] + ib_v[pl.ds(b0, 16)] + gval
            for j in range(D):
                u = plsc.load_gather(urows, [rows, par_u + j])
                v = plsc.load_gather(irows, [rows, par_i + j])
                acc = acc + u * v
            out_v[pl.ds(b0, 16)] = acc
            return 0

        lax.fori_loop(0, GPC, group_body, 0)

    pltpu.sync_copy(out_v, out_hbm.at[pl.ds(wid * BPW, BPW)])


def kernel(user_ids, item_ids, user_emb, item_emb, user_bias, item_bias, global_bias):
    uids = user_ids.astype(jnp.int32).reshape(B // CHUNK, CHUNK)
    iids = item_ids.astype(jnp.int32).reshape(B // CHUNK, CHUNK)
    uemb2 = user_emb.reshape(-1, 2 * D)
    iemb2 = item_emb.reshape(-1, 2 * D)
    gb16 = jnp.broadcast_to(global_bias.reshape(()), (16,))
    return _mf_kernel(uids, iids, uemb2, iemb2,
                      user_bias.reshape(-1), item_bias.reshape(-1),
                      gb16)


# speedup vs baseline: 1.3123x; 1.3123x over previous
import functools

import jax
import jax.numpy as jnp
from jax import lax
from jax.experimental import pallas as pl
from jax.experimental.pallas import tpu as pltpu
from jax.experimental.pallas import tpu_sc as plsc

B = 16384
D = 64
NC = 2
NS = 16
NW = NC * NS
CHUNK = 32
BPW = B // NW
CPW = BPW // CHUNK
GPC = CHUNK // 16

_mesh = plsc.VectorSubcoreMesh(core_axis_name="c", subcore_axis_name="s")


@functools.partial(
    pl.kernel,
    out_type=jax.ShapeDtypeStruct((B,), jnp.float32),
    mesh=_mesh,
    compiler_params=pltpu.CompilerParams(needs_layout_passes=False,
                                         use_tc_tiling_on_sc=True),
    scratch_types=[
        pltpu.VMEM((CPW, CHUNK), jnp.int32),
        pltpu.VMEM((CPW, CHUNK), jnp.int32),
        pltpu.VMEM((CPW, CHUNK), jnp.int32),
        pltpu.VMEM((CPW, CHUNK), jnp.int32),
        pltpu.VMEM((CHUNK * 8, D), jnp.float32),
        pltpu.VMEM((CHUNK * 8, D), jnp.float32),
        pltpu.VMEM((BPW,), jnp.float32),
        pltpu.VMEM((BPW,), jnp.float32),
        pltpu.VMEM((BPW,), jnp.float32),
        pltpu.VMEM((16,), jnp.float32),
        pltpu.SemaphoreType.DMA,
        pltpu.SemaphoreType.DMA,
    ],
)
def _mf_kernel(uid_hbm, iid_hbm, uemb, iemb, ubias, ibias, gbias, out_hbm,
               uidx_v, iidx_v, ugrp_v, igrp_v, urows, irows,
               ub_v, ib_v, out_v, gb_v, semg, semb):
    wid = lax.axis_index("s") * NC + lax.axis_index("c")
    crow0 = wid * CPW

    pltpu.sync_copy(uid_hbm.at[pl.ds(crow0, CPW)], uidx_v)
    pltpu.sync_copy(iid_hbm.at[pl.ds(crow0, CPW)], iidx_v)
    pltpu.sync_copy(gbias, gb_v)

    for c in range(CPW):
        for t in range(CHUNK // 16):
            s = pl.ds(t * 16, 16)
            ugrp_v[c, s] = lax.shift_left(
                lax.shift_right_logical(uidx_v[c, s], 3), 3)
            igrp_v[c, s] = lax.shift_left(
                lax.shift_right_logical(iidx_v[c, s], 3), 3)

    bias_copies = []
    for c in range(CPW):
        dst = pl.ds(c * CHUNK, CHUNK)
        bias_copies.append(pltpu.make_async_copy(ubias.at[uidx_v.at[c]],
                                                 ub_v.at[dst], semb))
        bias_copies.append(pltpu.make_async_copy(ibias.at[iidx_v.at[c]],
                                                 ib_v.at[dst], semb))
    for cp in bias_copies:
        cp.start()

    lane = lax.iota(jnp.int32, 16)
    gval = gb_v[...]

    for c in range(CPW):
        def fire(t, _, c=c):
            uvec = ugrp_v[c, pl.ds(t * 16, 16)]
            ivec = igrp_v[c, pl.ds(t * 16, 16)]
            for e in range(16):
                gu = pl.multiple_of(uvec[e], 8)
                gi = pl.multiple_of(ivec[e], 8)
                k8 = (t * 16 + e) * 8
                pltpu.make_async_copy(uemb.at[pl.ds(gu, 8)],
                                      urows.at[pl.ds(k8, 8)], semg).start()
                pltpu.make_async_copy(iemb.at[pl.ds(gi, 8)],
                                      irows.at[pl.ds(k8, 8)], semg).start()
            return 0

        lax.fori_loop(0, CHUNK // 16, fire, 0)

        def drain(k, _):
            pltpu.make_async_copy(uemb.at[pl.ds(0, 8)],
                                  urows.at[pl.ds(0, 8)], semg).wait()
            return 0

        lax.fori_loop(0, 2 * CHUNK, drain, 0)

        if c == 0:
            for cp in bias_copies:
                cp.wait()

        def group_body(g, _, c=c):
            pos = g * 16 + lane
            sel_u = uidx_v[c, pl.ds(g * 16, 16)] & 7
            sel_i = iidx_v[c, pl.ds(g * 16, 16)] & 7
            row_u = pos * 8 + sel_u
            row_i = pos * 8 + sel_i
            b0 = c * CHUNK + g * 16
            acc = ub_v[pl.ds(b0, 16)] + ib_v[pl.ds(b0, 16)] + gval
            for j in range(D):
                jv = jnp.full((16,), j, jnp.int32)
                u = plsc.load_gather(urows, [row_u, jv])
                v = plsc.load_gather(irows, [row_i, jv])
                acc = acc + u * v
            out_v[pl.ds(b0, 16)] = acc
            return 0

        lax.fori_loop(0, GPC, group_body, 0)

    pltpu.sync_copy(out_v, out_hbm.at[pl.ds(wid * BPW, BPW)])


def kernel(user_ids, item_ids, user_emb, item_emb, user_bias, item_bias, global_bias):
    uids = user_ids.astype(jnp.int32).reshape(B // CHUNK, CHUNK)
    iids = item_ids.astype(jnp.int32).reshape(B // CHUNK, CHUNK)
    gb16 = jnp.broadcast_to(global_bias.reshape(()), (16,))
    return _mf_kernel(uids, iids, user_emb, item_emb,
                      user_bias.reshape(-1), item_bias.reshape(-1),
                      gb16)
